# baseline (speedup 1.0000x reference)
import jax
import jax.numpy as jnp
from jax.experimental import pallas as pl

N = 10000
F = 128
BI1 = 400
BI2 = 400


def _support_kernel(x_ref, w_ref, s_ref):
    s_ref[...] = jnp.dot(
        x_ref[...].astype(jnp.bfloat16), w_ref[...],
        preferred_element_type=jnp.float32,
    ).astype(jnp.bfloat16)


def _layer1_kernel(adj_ref, s1_ref, b1_ref, w2_ref, a16_ref, h1_ref, s2_ref):
    a16 = adj_ref[...].astype(jnp.bfloat16)
    a16_ref[...] = a16
    z = jnp.dot(a16, s1_ref[...], preferred_element_type=jnp.float32) + b1_ref[...]
    h1 = jnp.maximum(z, 0.0)
    h1_ref[...] = h1
    s2_ref[...] = jnp.dot(
        h1.astype(jnp.bfloat16), w2_ref[...], preferred_element_type=jnp.float32
    ).astype(jnp.bfloat16)


def _layer2_kernel(a16_ref, s2_ref, b2_ref, h1_ref, w3_ref, s3_ref):
    z = jnp.dot(a16_ref[...], s2_ref[...], preferred_element_type=jnp.float32) + b2_ref[...]
    h2 = jnp.maximum(z, 0.0) + h1_ref[...]
    s3_ref[...] = jnp.dot(
        h2.astype(jnp.bfloat16), w3_ref[...], preferred_element_type=jnp.float32
    ).astype(jnp.bfloat16)


def _layer3_kernel(a16_ref, s3_ref, b3_ref, out_ref):
    out_ref[...] = (
        jnp.dot(a16_ref[...], s3_ref[...], preferred_element_type=jnp.float32)
        + b3_ref[...]
    )


def kernel(x, adj, W1, b1, W2, b2, W3, b3, interpret=False):
    w1 = W1.astype(jnp.bfloat16)
    w2 = W2.astype(jnp.bfloat16)
    w3 = W3.astype(jnp.bfloat16)
    b1r = b1.reshape(1, F)
    b2r = b2.reshape(1, F)
    b3r = b3.reshape(1, F)

    full = lambda shape: pl.BlockSpec(shape, lambda i: (0, 0))
    rows = lambda b, w: pl.BlockSpec((b, w), lambda i: (i, 0))

    s1 = pl.pallas_call(
        _support_kernel,
        grid=(1,),
        in_specs=[full((N, F)), full((F, F))],
        out_specs=full((N, F)),
        out_shape=jax.ShapeDtypeStruct((N, F), jnp.bfloat16),
        interpret=interpret,
    )(x, w1)

    a16, h1, s2 = pl.pallas_call(
        _layer1_kernel,
        grid=(N // BI1,),
        in_specs=[rows(BI1, N), full((N, F)), full((1, F)), full((F, F))],
        out_specs=[rows(BI1, N), rows(BI1, F), rows(BI1, F)],
        out_shape=[
            jax.ShapeDtypeStruct((N, N), jnp.bfloat16),
            jax.ShapeDtypeStruct((N, F), jnp.float32),
            jax.ShapeDtypeStruct((N, F), jnp.bfloat16),
        ],
        interpret=interpret,
    )(adj, s1, b1r, w2)

    s3 = pl.pallas_call(
        _layer2_kernel,
        grid=(N // BI2,),
        in_specs=[rows(BI2, N), full((N, F)), full((1, F)), rows(BI2, F), full((F, F))],
        out_specs=rows(BI2, F),
        out_shape=jax.ShapeDtypeStruct((N, F), jnp.bfloat16),
        interpret=interpret,
    )(a16, s2, b2r, h1, w3)

    out = pl.pallas_call(
        _layer3_kernel,
        grid=(N // BI2,),
        in_specs=[rows(BI2, N), full((N, F)), full((1, F))],
        out_specs=rows(BI2, F),
        out_shape=jax.ShapeDtypeStruct((N, F), jnp.float32),
        interpret=interpret,
    )(a16, s3, b3r)

    return out


# speedup vs baseline: 1.0583x; 1.0583x over previous
import jax
import jax.numpy as jnp
from jax.experimental import pallas as pl
from jax.experimental.pallas import tpu as pltpu

N = 10000
F = 128
BI1 = 400
BI2 = 1000
NB2 = N // BI2


def _support_kernel(x_ref, w_ref, s_ref):
    s_ref[...] = jnp.dot(
        x_ref[...].astype(jnp.bfloat16), w_ref[...],
        preferred_element_type=jnp.float32,
    ).astype(jnp.bfloat16)


def _layer1_kernel(adj_ref, s1_ref, b1_ref, w2_ref, a16_ref, h1_ref, s2_ref):
    a16 = adj_ref[...].astype(jnp.bfloat16)
    a16_ref[...] = a16
    z = jnp.dot(a16, s1_ref[...], preferred_element_type=jnp.float32) + b1_ref[...]
    h1 = jnp.maximum(z, 0.0)
    h1_ref[...] = h1
    s2_ref[...] = jnp.dot(
        h1.astype(jnp.bfloat16), w2_ref[...], preferred_element_type=jnp.float32
    ).astype(jnp.bfloat16)


def _layer23_kernel(a16_ref, s2_ref, b2_ref, h1_ref, w3_ref, b3_ref,
                    out_ref, s3_ref):
    t = pl.program_id(0)

    @pl.when(t < NB2)
    def _l2():
        z = jnp.dot(a16_ref[...], s2_ref[...], preferred_element_type=jnp.float32)
        h2 = jnp.maximum(z + b2_ref[...], 0.0) + h1_ref[...]
        s3_ref[pl.ds(t * BI2, BI2), :] = jnp.dot(
            h2.astype(jnp.bfloat16), w3_ref[...], preferred_element_type=jnp.float32
        ).astype(jnp.bfloat16)

    @pl.when(t >= NB2)
    def _l3():
        out_ref[...] = (
            jnp.dot(a16_ref[...], s3_ref[...], preferred_element_type=jnp.float32)
            + b3_ref[...]
        )


def kernel(x, adj, W1, b1, W2, b2, W3, b3, interpret=False):
    w1 = W1.astype(jnp.bfloat16)
    w2 = W2.astype(jnp.bfloat16)
    w3 = W3.astype(jnp.bfloat16)
    b1r = b1.reshape(1, F)
    b2r = b2.reshape(1, F)
    b3r = b3.reshape(1, F)

    full = lambda shape: pl.BlockSpec(shape, lambda i: (0, 0))
    rows = lambda b, w: pl.BlockSpec((b, w), lambda i: (i, 0))

    s1 = pl.pallas_call(
        _support_kernel,
        grid=(1,),
        in_specs=[full((N, F)), full((F, F))],
        out_specs=full((N, F)),
        out_shape=jax.ShapeDtypeStruct((N, F), jnp.bfloat16),
        interpret=interpret,
    )(x, w1)

    a16, h1, s2 = pl.pallas_call(
        _layer1_kernel,
        grid=(N // BI1,),
        in_specs=[rows(BI1, N), full((N, F)), full((1, F)), full((F, F))],
        out_specs=[rows(BI1, N), rows(BI1, F), rows(BI1, F)],
        out_shape=[
            jax.ShapeDtypeStruct((N, N), jnp.bfloat16),
            jax.ShapeDtypeStruct((N, F), jnp.float32),
            jax.ShapeDtypeStruct((N, F), jnp.bfloat16),
        ],
        interpret=interpret,
    )(adj, s1, b1r, w2)

    fwd_rev = lambda t: (jnp.where(t < NB2, t, 2 * NB2 - 1 - t), 0)
    out = pl.pallas_call(
        _layer23_kernel,
        grid=(2 * NB2,),
        in_specs=[
            pl.BlockSpec((BI2, N), fwd_rev),
            full((N, F)),
            full((1, F)),
            pl.BlockSpec((BI2, F), lambda t: (jnp.minimum(t, NB2 - 1), 0)),
            full((F, F)),
            full((1, F)),
        ],
        out_specs=pl.BlockSpec((BI2, F), fwd_rev),
        out_shape=jax.ShapeDtypeStruct((N, F), jnp.float32),
        scratch_shapes=[pltpu.VMEM((N, F), jnp.bfloat16)],
        interpret=interpret,
    )(a16, s2, b2r, h1, w3, b3r)

    return out


# speedup vs baseline: 1.0760x; 1.0168x over previous
import jax
import jax.numpy as jnp
from jax.experimental import pallas as pl
from jax.experimental.pallas import tpu as pltpu

N = 10000
F = 128
BI1 = 400
BI2 = 1000
NB2 = N // BI2


def _layer1_kernel(adj_ref, x_ref, w1_ref, b1_ref, w2_ref,
                   a16_ref, h1_ref, s2_ref, s1_ref):
    @pl.when(pl.program_id(0) == 0)
    def _s1():
        s1_ref[...] = jnp.dot(
            x_ref[...].astype(jnp.bfloat16), w1_ref[...],
            preferred_element_type=jnp.float32,
        ).astype(jnp.bfloat16)

    a16 = adj_ref[...].astype(jnp.bfloat16)
    a16_ref[...] = a16
    z = jnp.dot(a16, s1_ref[...], preferred_element_type=jnp.float32) + b1_ref[...]
    h1 = jnp.maximum(z, 0.0)
    h1b = h1.astype(jnp.bfloat16)
    h1_ref[...] = h1b
    s2_ref[...] = jnp.dot(
        h1b, w2_ref[...], preferred_element_type=jnp.float32
    ).astype(jnp.bfloat16)


def _layer23_kernel(a16_ref, s2_ref, b2_ref, h1_ref, w3_ref, b3_ref,
                    out_ref, s3_ref):
    t = pl.program_id(0)

    @pl.when(t < NB2)
    def _l2():
        z = jnp.dot(a16_ref[...], s2_ref[...], preferred_element_type=jnp.float32)
        h2 = jnp.maximum(z + b2_ref[...], 0.0) + h1_ref[...].astype(jnp.float32)
        s3_ref[pl.ds(t * BI2, BI2), :] = jnp.dot(
            h2.astype(jnp.bfloat16), w3_ref[...], preferred_element_type=jnp.float32
        ).astype(jnp.bfloat16)

    @pl.when(t >= NB2)
    def _l3():
        out_ref[...] = (
            jnp.dot(a16_ref[...], s3_ref[...], preferred_element_type=jnp.float32)
            + b3_ref[...]
        )


def kernel(x, adj, W1, b1, W2, b2, W3, b3, interpret=False):
    w1 = W1.astype(jnp.bfloat16)
    w2 = W2.astype(jnp.bfloat16)
    w3 = W3.astype(jnp.bfloat16)
    b1r = b1.reshape(1, F)
    b2r = b2.reshape(1, F)
    b3r = b3.reshape(1, F)

    full = lambda shape: pl.BlockSpec(shape, lambda i: (0, 0))
    rows = lambda b, w: pl.BlockSpec((b, w), lambda i: (i, 0))

    a16, h1, s2 = pl.pallas_call(
        _layer1_kernel,
        grid=(N // BI1,),
        in_specs=[rows(BI1, N), full((N, F)), full((F, F)), full((1, F)), full((F, F))],
        out_specs=[rows(BI1, N), rows(BI1, F), rows(BI1, F)],
        out_shape=[
            jax.ShapeDtypeStruct((N, N), jnp.bfloat16),
            jax.ShapeDtypeStruct((N, F), jnp.bfloat16),
            jax.ShapeDtypeStruct((N, F), jnp.bfloat16),
        ],
        scratch_shapes=[pltpu.VMEM((N, F), jnp.bfloat16)],
        interpret=interpret,
    )(adj, x, w1, b1r, w2)

    fwd_rev = lambda t: (jnp.where(t < NB2, t, 2 * NB2 - 1 - t), 0)
    out = pl.pallas_call(
        _layer23_kernel,
        grid=(2 * NB2,),
        in_specs=[
            pl.BlockSpec((BI2, N), fwd_rev),
            full((N, F)),
            full((1, F)),
            pl.BlockSpec((BI2, F), lambda t: (jnp.minimum(t, NB2 - 1), 0)),
            full((F, F)),
            full((1, F)),
        ],
        out_specs=pl.BlockSpec(
            (BI2, F), lambda t: (jnp.where(t < NB2, NB2 - 1, 2 * NB2 - 1 - t), 0)
        ),
        out_shape=jax.ShapeDtypeStruct((N, F), jnp.float32),
        scratch_shapes=[pltpu.VMEM((N, F), jnp.bfloat16)],
        interpret=interpret,
    )(a16, s2, b2r, h1, w3, b3r)

    return out


# speedup vs baseline: 1.0938x; 1.0165x over previous
import jax
import jax.numpy as jnp
from jax.experimental import pallas as pl
from jax.experimental.pallas import tpu as pltpu

N = 10000
F = 128
BI1 = 400
BI2 = 1000
NB2 = N // BI2


def _layer1_kernel(adj_ref, x_ref, w1_ref, b1_ref, w2_ref,
                   a16_ref, h1_ref, s2_ref, s1_ref):
    @pl.when(pl.program_id(0) == 0)
    def _s1():
        s1_ref[...] = jnp.dot(
            x_ref[...].astype(jnp.bfloat16), w1_ref[...].astype(jnp.bfloat16),
            preferred_element_type=jnp.float32,
        ).astype(jnp.bfloat16)

    a16 = adj_ref[...].astype(jnp.bfloat16)
    a16_ref[...] = a16
    z = jnp.dot(a16, s1_ref[...], preferred_element_type=jnp.float32) + b1_ref[...]
    h1 = jnp.maximum(z, 0.0)
    h1b = h1.astype(jnp.bfloat16)
    h1_ref[...] = h1b
    s2_ref[...] = jnp.dot(
        h1b, w2_ref[...].astype(jnp.bfloat16), preferred_element_type=jnp.float32
    ).astype(jnp.bfloat16)


def _layer23_kernel(a16_ref, s2_ref, b2_ref, h1_ref, w3_ref, b3_ref,
                    out_ref, s3_ref):
    t = pl.program_id(0)

    @pl.when(t < NB2)
    def _l2():
        z = jnp.dot(a16_ref[...], s2_ref[...], preferred_element_type=jnp.float32)
        h2 = jnp.maximum(z + b2_ref[...], 0.0) + h1_ref[...].astype(jnp.float32)
        s3_ref[pl.ds(t * BI2, BI2), :] = jnp.dot(
            h2.astype(jnp.bfloat16), w3_ref[...].astype(jnp.bfloat16),
            preferred_element_type=jnp.float32,
        ).astype(jnp.bfloat16)

    @pl.when(t >= NB2)
    def _l3():
        out_ref[...] = (
            jnp.dot(a16_ref[...], s3_ref[...], preferred_element_type=jnp.float32)
            + b3_ref[...]
        )


def kernel(x, adj, W1, b1, W2, b2, W3, b3, interpret=False):
    b1r = b1.reshape(1, F)
    b2r = b2.reshape(1, F)
    b3r = b3.reshape(1, F)

    full = lambda shape: pl.BlockSpec(shape, lambda i: (0, 0))
    rows = lambda b, w: pl.BlockSpec((b, w), lambda i: (i, 0))

    a16, h1, s2 = pl.pallas_call(
        _layer1_kernel,
        grid=(N // BI1,),
        in_specs=[rows(BI1, N), full((N, F)), full((F, F)), full((1, F)), full((F, F))],
        out_specs=[rows(BI1, N), rows(BI1, F), rows(BI1, F)],
        out_shape=[
            jax.ShapeDtypeStruct((N, N), jnp.bfloat16),
            jax.ShapeDtypeStruct((N, F), jnp.bfloat16),
            jax.ShapeDtypeStruct((N, F), jnp.bfloat16),
        ],
        scratch_shapes=[pltpu.VMEM((N, F), jnp.bfloat16)],
        interpret=interpret,
    )(adj, x, W1, b1r, W2)

    fwd_rev = lambda t: (jnp.where(t < NB2, t, 2 * NB2 - 1 - t), 0)
    out = pl.pallas_call(
        _layer23_kernel,
        grid=(2 * NB2,),
        in_specs=[
            pl.BlockSpec((BI2, N), fwd_rev),
            full((N, F)),
            full((1, F)),
            pl.BlockSpec((BI2, F), lambda t: (jnp.minimum(t, NB2 - 1), 0)),
            full((F, F)),
            full((1, F)),
        ],
        out_specs=pl.BlockSpec(
            (BI2, F), lambda t: (jnp.where(t < NB2, NB2 - 1, 2 * NB2 - 1 - t), 0)
        ),
        out_shape=jax.ShapeDtypeStruct((N, F), jnp.float32),
        scratch_shapes=[pltpu.VMEM((N, F), jnp.bfloat16)],
        interpret=interpret,
    )(a16, s2, b2r, h1, W3, b3r)

    return out
